# baseline (speedup 1.0000x reference)
import jax
import jax.numpy as jnp
from jax.experimental import pallas as pl
from jax.experimental.pallas import tpu as pltpu

G = 4
K = 1024
D = 32
DP = 40
B = 8
C = 128
T = 2048
TT = 2048
NT = T // TT
N_TOK = B * T
_INV_ELEMS = 1.0 / (N_TOK * D)
_INV_NTOK = 1.0 / N_TOK


def _vq_kernel(cbm_ref, cba_ref, x_ref, y_ref, stats_ref, counts_scr, loss_scr):
    g = pl.program_id(0)
    b = pl.program_id(1)
    t = pl.program_id(2)
    nb = pl.num_programs(1)
    nt = pl.num_programs(2)

    @pl.when((b == 0) & (t == 0))
    def _():
        counts_scr[...] = jnp.zeros_like(counts_scr)

    @pl.when((g == 0) & (b == 0) & (t == 0))
    def _():
        loss_scr[0, 0] = 0.0
        stats_ref[...] = jnp.zeros_like(stats_ref)

    cbm = cbm_ref[0]
    cba = cba_ref[0]
    xb = x_ref[0, 0]

    S = jax.lax.dot_general(cbm, xb, (((1,), (0,)), ((), ())),
                            preferred_element_type=jnp.float32)
    S = S + cba[:, D + 1:D + 2]
    minval = jnp.min(S, axis=0, keepdims=True)
    matches = (S == minval).astype(jnp.float32)

    xq_aug = jax.lax.dot_general(cba, matches, (((0,), (0,)), ((), ())),
                                 preferred_element_type=jnp.float32)
    nm = xq_aug[D:D + 1, :]
    xq = xq_aug[0:D, :] * (1.0 / nm)
    y_ref[0, 0] = xq

    diff = xb - xq
    loss_scr[0, 0] += jnp.sum(diff * diff)
    counts_scr[...] += jnp.sum(matches, axis=1, keepdims=True)

    last_in_group = (b == nb - 1) & (t == nt - 1)
    rows = jax.lax.broadcasted_iota(jnp.int32, (8, 128), 0)

    @pl.when(last_in_group)
    def _():
        probs = counts_scr[...] * _INV_NTOK
        ent = jnp.sum(probs * jnp.log(probs + 1e-10))
        pp = jnp.exp(-ent)
        stats_ref[...] = jnp.where(rows == g, pp, stats_ref[...])

    @pl.when(last_in_group & (g == pl.num_programs(0) - 1))
    def _():
        loss = loss_scr[0, 0] * _INV_ELEMS
        stats_ref[...] = jnp.where(rows == G, loss, stats_ref[...])


def kernel(x, codebook_0, codebook_1, codebook_2, codebook_3):
    cbs = jnp.stack([codebook_0, codebook_1, codebook_2, codebook_3], axis=0)
    x4 = x.reshape(B, G, D, T)

    cbsq = jnp.sum(cbs * cbs, axis=2, keepdims=True)
    zpad = jnp.zeros((G, K, DP - D - 2), jnp.float32)
    cbms = -2.0 * cbs
    cbas = jnp.concatenate(
        [cbs, jnp.ones((G, K, 1), jnp.float32), cbsq, zpad], axis=2)

    y4, stats = pl.pallas_call(
        _vq_kernel,
        grid=(G, B, NT),
        in_specs=[
            pl.BlockSpec((1, K, D), lambda g, b, t: (g, 0, 0)),
            pl.BlockSpec((1, K, DP), lambda g, b, t: (g, 0, 0)),
            pl.BlockSpec((1, 1, D, TT), lambda g, b, t: (b, g, 0, t)),
        ],
        out_specs=[
            pl.BlockSpec((1, 1, D, TT), lambda g, b, t: (b, g, 0, t)),
            pl.BlockSpec((8, 128), lambda g, b, t: (0, 0)),
        ],
        out_shape=[
            jax.ShapeDtypeStruct((B, G, D, T), jnp.float32),
            jax.ShapeDtypeStruct((8, 128), jnp.float32),
        ],
        scratch_shapes=[
            pltpu.VMEM((K, 1), jnp.float32),
            pltpu.SMEM((1, 1), jnp.float32),
        ],
        compiler_params=pltpu.CompilerParams(
            dimension_semantics=("arbitrary", "arbitrary", "arbitrary"),
        ),
    )(cbms, cbas, x4)

    return y4.reshape(B, C, T), stats[G, 0], stats[0:G, 0]


# speedup vs baseline: 1.0934x; 1.0934x over previous
import jax
import jax.numpy as jnp
from jax.experimental import pallas as pl
from jax.experimental.pallas import tpu as pltpu

G = 4
K = 1024
D = 32
DP = 40
B = 8
C = 128
T = 2048
TT = 2048
NT = T // TT
NC = 2
TC = TT // NC
N_TOK = B * T
_INV_ELEMS = 1.0 / (N_TOK * D)
_INV_NTOK = 1.0 / N_TOK


def _vq_kernel(cbm_ref, cba_ref, x_ref, y_ref, stats_ref, counts_scr, loss_scr):
    g = pl.program_id(0)
    b = pl.program_id(1)
    t = pl.program_id(2)
    nb = pl.num_programs(1)
    nt = pl.num_programs(2)

    @pl.when((b == 0) & (t == 0))
    def _():
        counts_scr[...] = jnp.zeros_like(counts_scr)

    @pl.when((g == 0) & (b == 0) & (t == 0))
    def _():
        loss_scr[0, 0] = 0.0
        stats_ref[...] = jnp.zeros_like(stats_ref)

    cbm = cbm_ref[0]
    cba = cba_ref[0]
    cbsq_col = cba[:, D + 1:D + 2]

    for c in range(NC):
        sl = pl.ds(c * TC, TC)
        xb = x_ref[0, 0, :, sl]

        S = jax.lax.dot_general(cbm, xb, (((1,), (0,)), ((), ())),
                                preferred_element_type=jnp.float32)
        S = S + cbsq_col
        minval = jnp.min(S, axis=0, keepdims=True)
        matches = (S == minval).astype(jnp.float32)

        xq_aug = jax.lax.dot_general(cba, matches, (((0,), (0,)), ((), ())),
                                     preferred_element_type=jnp.float32)
        nm = xq_aug[D:D + 1, :]
        xq = xq_aug[0:D, :] * (1.0 / nm)
        y_ref[0, 0, :, sl] = xq

        diff = xb - xq
        loss_scr[0, 0] += jnp.sum(diff * diff)
        counts_scr[...] += jnp.sum(matches, axis=1, keepdims=True)

    last_in_group = (b == nb - 1) & (t == nt - 1)
    rows = jax.lax.broadcasted_iota(jnp.int32, (8, 128), 0)

    @pl.when(last_in_group)
    def _():
        probs = counts_scr[...] * _INV_NTOK
        ent = jnp.sum(probs * jnp.log(probs + 1e-10))
        pp = jnp.exp(-ent)
        stats_ref[...] = jnp.where(rows == g, pp, stats_ref[...])

    @pl.when(last_in_group & (g == pl.num_programs(0) - 1))
    def _():
        loss = loss_scr[0, 0] * _INV_ELEMS
        stats_ref[...] = jnp.where(rows == G, loss, stats_ref[...])


def kernel(x, codebook_0, codebook_1, codebook_2, codebook_3):
    cbs = jnp.stack([codebook_0, codebook_1, codebook_2, codebook_3], axis=0)
    x4 = x.reshape(B, G, D, T)

    cbsq = jnp.sum(cbs * cbs, axis=2, keepdims=True)
    zpad = jnp.zeros((G, K, DP - D - 2), jnp.float32)
    cbms = -2.0 * cbs
    cbas = jnp.concatenate(
        [cbs, jnp.ones((G, K, 1), jnp.float32), cbsq, zpad], axis=2)

    y4, stats = pl.pallas_call(
        _vq_kernel,
        grid=(G, B, NT),
        in_specs=[
            pl.BlockSpec((1, K, D), lambda g, b, t: (g, 0, 0)),
            pl.BlockSpec((1, K, DP), lambda g, b, t: (g, 0, 0)),
            pl.BlockSpec((1, 1, D, TT), lambda g, b, t: (b, g, 0, t)),
        ],
        out_specs=[
            pl.BlockSpec((1, 1, D, TT), lambda g, b, t: (b, g, 0, t)),
            pl.BlockSpec((8, 128), lambda g, b, t: (0, 0)),
        ],
        out_shape=[
            jax.ShapeDtypeStruct((B, G, D, T), jnp.float32),
            jax.ShapeDtypeStruct((8, 128), jnp.float32),
        ],
        scratch_shapes=[
            pltpu.VMEM((K, 1), jnp.float32),
            pltpu.SMEM((1, 1), jnp.float32),
        ],
        compiler_params=pltpu.CompilerParams(
            dimension_semantics=("arbitrary", "arbitrary", "arbitrary"),
        ),
    )(cbms, cbas, x4)

    return y4.reshape(B, C, T), stats[G, 0], stats[0:G, 0]


# speedup vs baseline: 1.1629x; 1.0635x over previous
import jax
import jax.numpy as jnp
from jax.experimental import pallas as pl
from jax.experimental.pallas import tpu as pltpu

G = 4
K = 1024
D = 32
DP = 40
B = 8
C = 128
T = 2048
TT = 2048
NT = T // TT
NC = 2
TC = TT // NC
BB = 2
N_TOK = B * T
_INV_ELEMS = 1.0 / (N_TOK * D)
_INV_NTOK = 1.0 / N_TOK


def _vq_kernel(cbm_ref, cba_ref, x_ref, y_ref, stats_ref, counts_scr, loss_scr):
    g = pl.program_id(0)
    b = pl.program_id(1)
    t = pl.program_id(2)
    nb = pl.num_programs(1)
    nt = pl.num_programs(2)

    @pl.when((b == 0) & (t == 0))
    def _():
        counts_scr[...] = jnp.zeros_like(counts_scr)

    @pl.when((g == 0) & (b == 0) & (t == 0))
    def _():
        loss_scr[0, 0] = 0.0
        stats_ref[...] = jnp.zeros_like(stats_ref)

    cbm = cbm_ref[0]
    cba = cba_ref[0]
    cbsq_col = cba[:, D + 1:D + 2]

    for bi in range(BB):
      for c in range(NC):
        sl = pl.ds(c * TC, TC)
        xb = x_ref[bi, 0, :, sl]

        S = jax.lax.dot_general(cbm, xb, (((1,), (0,)), ((), ())),
                                preferred_element_type=jnp.float32)
        S = S + cbsq_col
        minval = jnp.min(S, axis=0, keepdims=True)
        matches = (S == minval).astype(jnp.float32)

        xq_aug = jax.lax.dot_general(cba, matches, (((0,), (0,)), ((), ())),
                                     preferred_element_type=jnp.float32)
        nm = xq_aug[D:D + 1, :]
        xq = xq_aug[0:D, :] * (1.0 / nm)
        y_ref[bi, 0, :, sl] = xq

        diff = xb - xq
        loss_scr[0, 0] += jnp.sum(diff * diff)
        counts_scr[...] += jnp.sum(matches, axis=1, keepdims=True)

    last_in_group = (b == nb - 1) & (t == nt - 1)
    rows = jax.lax.broadcasted_iota(jnp.int32, (8, 128), 0)

    @pl.when(last_in_group)
    def _():
        probs = counts_scr[...] * _INV_NTOK
        ent = jnp.sum(probs * jnp.log(probs + 1e-10))
        pp = jnp.exp(-ent)
        stats_ref[...] = jnp.where(rows == g, pp, stats_ref[...])

    @pl.when(last_in_group & (g == pl.num_programs(0) - 1))
    def _():
        loss = loss_scr[0, 0] * _INV_ELEMS
        stats_ref[...] = jnp.where(rows == G, loss, stats_ref[...])


def kernel(x, codebook_0, codebook_1, codebook_2, codebook_3):
    cbs = jnp.stack([codebook_0, codebook_1, codebook_2, codebook_3], axis=0)
    x4 = x.reshape(B, G, D, T)

    cbsq = jnp.sum(cbs * cbs, axis=2, keepdims=True)
    zpad = jnp.zeros((G, K, DP - D - 2), jnp.float32)
    cbms = -2.0 * cbs
    cbas = jnp.concatenate(
        [cbs, jnp.ones((G, K, 1), jnp.float32), cbsq, zpad], axis=2)

    y4, stats = pl.pallas_call(
        _vq_kernel,
        grid=(G, B // BB, NT),
        in_specs=[
            pl.BlockSpec((1, K, D), lambda g, b, t: (g, 0, 0)),
            pl.BlockSpec((1, K, DP), lambda g, b, t: (g, 0, 0)),
            pl.BlockSpec((BB, 1, D, TT), lambda g, b, t: (b, g, 0, t)),
        ],
        out_specs=[
            pl.BlockSpec((BB, 1, D, TT), lambda g, b, t: (b, g, 0, t)),
            pl.BlockSpec((8, 128), lambda g, b, t: (0, 0)),
        ],
        out_shape=[
            jax.ShapeDtypeStruct((B, G, D, T), jnp.float32),
            jax.ShapeDtypeStruct((8, 128), jnp.float32),
        ],
        scratch_shapes=[
            pltpu.VMEM((K, 1), jnp.float32),
            pltpu.SMEM((1, 1), jnp.float32),
        ],
        compiler_params=pltpu.CompilerParams(
            dimension_semantics=("arbitrary", "arbitrary", "arbitrary"),
        ),
    )(cbms, cbas, x4)

    return y4.reshape(B, C, T), stats[G, 0], stats[0:G, 0]
